# Initial kernel scaffold; baseline (speedup 1.0000x reference)
import jax
import jax.numpy as jnp
from jax.experimental import pallas as pl


def kernel(x, edge_index, W1, b1, W2, b2):
    raise NotImplementedError("write your pallas kernel here")


# speedup vs baseline: 14.7937x; 14.7937x over previous
import functools

import jax
import jax.numpy as jnp
from jax import lax
from jax.experimental import pallas as pl
from jax.experimental.pallas import tpu as pltpu
from jax.experimental.pallas import tpu_sc as plsc

N = 10000
E = 320000
D = 128

NS = 16
EW = E // NS
K = 40
NB = 5
NBATCH = EW // (NB * K)
HALF = NBATCH // 2
STRIPE = 632
NPAD = NS * STRIPE

_mesh = plsc.VectorSubcoreMesh(
    core_axis_name="c", subcore_axis_name="s", num_cores=1, num_subcores=NS
)


@functools.partial(
    pl.kernel,
    out_type=jax.ShapeDtypeStruct((N,), jnp.float32),
    mesh=_mesh,
    scratch_types=[
        pltpu.VMEM((2, NB, K), jnp.int32),
        pltpu.VMEM((K,), jnp.float32),
        pltpu.VMEM_SHARED((N,), jnp.float32),
        pltpu.SemaphoreType.DMA,
        pltpu.SemaphoreType.DMA,
        pltpu.SemaphoreType.DMA,
    ],
)
def _deg_kernel(dst_hbm, zeros1_hbm, ones_hbm, out_hbm, dstb, onesv, accum,
                si0, si1, ss):
    s = lax.axis_index("s")
    si = (si0, si1)

    @pl.when(s == 0)
    def _zero():
        pltpu.sync_copy(zeros1_hbm, accum)

    pltpu.sync_copy(ones_hbm, onesv)
    pltpu.async_copy(dst_hbm.at[s, 0], dstb.at[0], si0)
    pltpu.async_copy(dst_hbm.at[s, 1], dstb.at[1], si1)
    plsc.subcore_barrier()

    def process(j, p):
        pltpu.make_async_copy(dst_hbm.at[s, j], dstb.at[p], si[p]).wait()
        cps = [
            pltpu.async_copy(onesv, accum.at[dstb.at[p, b]], ss, add=True)
            for b in range(NB)
        ]
        for cp in cps:
            cp.wait()

    def body(i, carry):
        process(2 * i, 0)

        @pl.when(i < HALF - 1)
        def _pf0():
            pltpu.async_copy(dst_hbm.at[s, 2 * i + 2], dstb.at[0], si0)

        process(2 * i + 1, 1)

        @pl.when(i < HALF - 1)
        def _pf1():
            pltpu.async_copy(dst_hbm.at[s, 2 * i + 3], dstb.at[1], si1)

        return carry

    lax.fori_loop(0, HALF, body, 0)
    plsc.subcore_barrier()

    @pl.when(s == 0)
    def _flush():
        pltpu.sync_copy(accum, out_hbm)


@functools.partial(
    pl.kernel,
    out_type=jax.ShapeDtypeStruct((NPAD, D), jnp.float32),
    mesh=_mesh,
    scratch_types=[
        pltpu.VMEM((2, NB, K), jnp.int32),
        pltpu.VMEM((2, NB, K), jnp.int32),
        pltpu.VMEM((NB, K, D), jnp.float32),
        pltpu.VMEM_SHARED((NPAD, D), jnp.float32),
        pltpu.SemaphoreType.DMA,
        pltpu.SemaphoreType.DMA,
        pltpu.SemaphoreType.DMA,
        pltpu.SemaphoreType.DMA,
        pltpu.SemaphoreType.DMA,
        pltpu.SemaphoreType.DMA,
        pltpu.SemaphoreType.DMA,
        pltpu.SemaphoreType.DMA,
        pltpu.SemaphoreType.DMA,
    ],
)
def _agg_kernel(g_hbm, src_hbm, dst_hbm, zeros2_hbm, out_hbm,
                srcb, dstb, rows, accum,
                semz, si0, si1, sg0, sg1, sg2, sg3, sg4, ss):
    s = lax.axis_index("s")
    si = (si0, si1)
    sg = (sg0, sg1, sg2, sg3, sg4)
    row0 = s * STRIPE

    zcp = pltpu.async_copy(
        zeros2_hbm.at[pl.ds(row0, STRIPE)], accum.at[pl.ds(row0, STRIPE)], semz
    )
    pltpu.async_copy(src_hbm.at[s, 0], srcb.at[0], si0)
    pltpu.async_copy(dst_hbm.at[s, 0], dstb.at[0], si0)
    pltpu.async_copy(src_hbm.at[s, 1], srcb.at[1], si1)
    pltpu.async_copy(dst_hbm.at[s, 1], dstb.at[1], si1)
    zcp.wait()
    plsc.subcore_barrier()

    def process(j, p):
        pltpu.make_async_copy(src_hbm.at[s, j], srcb.at[p], si[p]).wait()
        pltpu.make_async_copy(dst_hbm.at[s, j], dstb.at[p], si[p]).wait()
        gcps = [
            pltpu.async_copy(g_hbm.at[srcb.at[p, b]], rows.at[b], sg[b])
            for b in range(NB)
        ]
        scps = []
        for b in range(NB):
            gcps[b].wait()
            scps.append(
                pltpu.async_copy(
                    rows.at[b], accum.at[dstb.at[p, b]], ss, add=True
                )
            )
        for cp in scps:
            cp.wait()

    def body(i, carry):
        process(2 * i, 0)

        @pl.when(i < HALF - 1)
        def _pf0():
            pltpu.async_copy(src_hbm.at[s, 2 * i + 2], srcb.at[0], si0)
            pltpu.async_copy(dst_hbm.at[s, 2 * i + 2], dstb.at[0], si0)

        process(2 * i + 1, 1)

        @pl.when(i < HALF - 1)
        def _pf1():
            pltpu.async_copy(src_hbm.at[s, 2 * i + 3], srcb.at[1], si1)
            pltpu.async_copy(dst_hbm.at[s, 2 * i + 3], dstb.at[1], si1)

        return carry

    lax.fori_loop(0, HALF, body, 0)
    plsc.subcore_barrier()

    pltpu.sync_copy(
        accum.at[pl.ds(row0, STRIPE)], out_hbm.at[pl.ds(row0, STRIPE)]
    )


def _tc_a_body(x_ref, w_ref, deg_ref, h_ref, g_ref, dinv_ref):
    h = jnp.dot(x_ref[...], w_ref[...], preferred_element_type=jnp.float32)
    dinv = lax.rsqrt(deg_ref[...] + 1.0)
    h_ref[...] = h
    g_ref[...] = h * dinv
    dinv_ref[...] = dinv


_tc_a = pl.pallas_call(
    _tc_a_body,
    out_shape=(
        jax.ShapeDtypeStruct((N, D), jnp.float32),
        jax.ShapeDtypeStruct((N, D), jnp.float32),
        jax.ShapeDtypeStruct((N, 1), jnp.float32),
    ),
)


def _tc_mid_body(p_ref, h_ref, dinv_ref, b_ref, w_ref, h2_ref, g2_ref, y_ref):
    dinv = dinv_ref[...]
    pre = dinv * (p_ref[:N, :] + dinv * h_ref[...]) + b_ref[...][None, :]
    y_ref[...] = pre
    z = jnp.maximum(pre, 0.0)
    h2 = jnp.dot(z, w_ref[...], preferred_element_type=jnp.float32)
    h2_ref[...] = h2
    g2_ref[...] = h2 * dinv


_tc_mid = pl.pallas_call(
    _tc_mid_body,
    out_shape=(
        jax.ShapeDtypeStruct((N, D), jnp.float32),
        jax.ShapeDtypeStruct((N, D), jnp.float32),
        jax.ShapeDtypeStruct((N, D), jnp.float32),
    ),
)


@jax.jit
def kernel(x, edge_index, W1, b1, W2, b2):
    ei = edge_index.astype(jnp.int32)
    src4 = ei[0].reshape(NS, NBATCH, NB, K)
    dst4 = ei[1].reshape(NS, NBATCH, NB, K)
    zeros1 = jnp.zeros((N,), jnp.float32)
    zeros2 = jnp.zeros((NPAD, D), jnp.float32)

    ones = jnp.ones((K,), jnp.float32)
    deg = _deg_kernel(dst4, zeros1, ones)
    degc = deg[:, None]

    h1, g1, dinv = _tc_a(x, W1, degc)

    def body(carry, wb):
        h, g = carry
        w, b = wb
        p = _agg_kernel(g, src4, dst4, zeros2)
        h2, g2, y = _tc_mid(p, h, dinv, b, w)
        return (h2, g2), y

    _, ys = lax.scan(body, (h1, g1), (jnp.stack([W2, W2]), jnp.stack([b1, b2])))
    return ys[1]


# speedup vs baseline: 18.2301x; 1.2323x over previous
import functools

import jax
import jax.numpy as jnp
from jax import lax
from jax.experimental import pallas as pl
from jax.experimental.pallas import tpu as pltpu
from jax.experimental.pallas import tpu_sc as plsc

N = 10000
E = 320000
D = 128

NS = 16
EW = E // NS
K = 40
NB = 5
NBATCH = EW // (NB * K)
HALF = NBATCH // 2
STRIPE = 632
NPAD = NS * STRIPE

_mesh = plsc.VectorSubcoreMesh(
    core_axis_name="c", subcore_axis_name="s", num_cores=1, num_subcores=NS
)


@functools.partial(
    pl.kernel,
    out_type=jax.ShapeDtypeStruct((N,), jnp.float32),
    mesh=_mesh,
    scratch_types=[
        pltpu.VMEM((2, NB, K), jnp.int32),
        pltpu.VMEM((K,), jnp.float32),
        pltpu.VMEM_SHARED((N,), jnp.float32),
        pltpu.SemaphoreType.DMA,
        pltpu.SemaphoreType.DMA,
        pltpu.SemaphoreType.DMA,
    ],
)
def _deg_kernel(dst_hbm, zeros1_hbm, ones_hbm, out_hbm, dstb, onesv, accum,
                si0, si1, ss):
    s = lax.axis_index("s")
    si = (si0, si1)

    @pl.when(s == 0)
    def _zero():
        pltpu.sync_copy(zeros1_hbm, accum)

    pltpu.sync_copy(ones_hbm, onesv)
    pltpu.async_copy(dst_hbm.at[s, 0], dstb.at[0], si0)
    pltpu.async_copy(dst_hbm.at[s, 1], dstb.at[1], si1)
    plsc.subcore_barrier()

    def process(j, p):
        pltpu.make_async_copy(dst_hbm.at[s, j], dstb.at[p], si[p]).wait()
        cps = [
            pltpu.async_copy(onesv, accum.at[dstb.at[p, b]], ss, add=True)
            for b in range(NB)
        ]
        for cp in cps:
            cp.wait()

    def body(i, carry):
        process(2 * i, 0)

        @pl.when(i < HALF - 1)
        def _pf0():
            pltpu.async_copy(dst_hbm.at[s, 2 * i + 2], dstb.at[0], si0)

        process(2 * i + 1, 1)

        @pl.when(i < HALF - 1)
        def _pf1():
            pltpu.async_copy(dst_hbm.at[s, 2 * i + 3], dstb.at[1], si1)

        return carry

    lax.fori_loop(0, HALF, body, 0)
    plsc.subcore_barrier()

    @pl.when(s == 0)
    def _flush():
        pltpu.sync_copy(accum, out_hbm)


@functools.partial(
    pl.kernel,
    out_type=jax.ShapeDtypeStruct((NPAD, D), jnp.float32),
    mesh=_mesh,
    scratch_types=[
        pltpu.VMEM((4, NB, K), jnp.int32),
        pltpu.VMEM((4, NB, K), jnp.int32),
        pltpu.VMEM((NB, K, D), jnp.float32),
        pltpu.VMEM_SHARED((NPAD, D), jnp.float32),
        pltpu.SemaphoreType.DMA,
        pltpu.SemaphoreType.DMA,
        pltpu.SemaphoreType.DMA,
        pltpu.SemaphoreType.DMA,
        pltpu.SemaphoreType.DMA,
        pltpu.SemaphoreType.DMA,
        pltpu.SemaphoreType.DMA,
        pltpu.SemaphoreType.DMA,
        pltpu.SemaphoreType.DMA,
        pltpu.SemaphoreType.DMA,
        pltpu.SemaphoreType.DMA,
        pltpu.SemaphoreType.DMA,
        pltpu.SemaphoreType.DMA,
        pltpu.SemaphoreType.DMA,
        pltpu.SemaphoreType.DMA,
    ],
)
def _agg_kernel(g_hbm, src_hbm, dst_hbm, zeros2_hbm, out_hbm,
                srcb, dstb, rows, accum,
                semz, si0, si1, si2, si3, sg0, sg1, sg2, sg3, sg4,
                ss0, ss1, ss2, ss3, ss4):
    s = lax.axis_index("s")
    si = (si0, si1, si2, si3)
    sg = (sg0, sg1, sg2, sg3, sg4)
    ssb = (ss0, ss1, ss2, ss3, ss4)
    row0 = s * STRIPE

    zcp = pltpu.async_copy(
        zeros2_hbm.at[pl.ds(row0, STRIPE)], accum.at[pl.ds(row0, STRIPE)], semz
    )
    for t in range(3):
        pltpu.async_copy(src_hbm.at[s, t], srcb.at[t], si[t])
        pltpu.async_copy(dst_hbm.at[s, t], dstb.at[t], si[t])
    zcp.wait()
    plsc.subcore_barrier()

    def scat_wait(b):
        pltpu.make_async_copy(
            rows.at[b], accum.at[dstb.at[0, b]], ssb[b]
        ).wait()

    def prefetch(j, t):
        pltpu.async_copy(src_hbm.at[s, j], srcb.at[t], si[t])
        pltpu.async_copy(dst_hbm.at[s, j], dstb.at[t], si[t])

    def process(j, t, wait_pred):
        pltpu.make_async_copy(src_hbm.at[s, j], srcb.at[t], si[t]).wait()
        pltpu.make_async_copy(dst_hbm.at[s, j], dstb.at[t], si[t]).wait()
        gcps = []
        for b in range(NB):
            if wait_pred is None:
                scat_wait(b)
            else:
                @pl.when(wait_pred)
                def _w(b=b):
                    scat_wait(b)
            gcps.append(
                pltpu.async_copy(g_hbm.at[srcb.at[t, b]], rows.at[b], sg[b])
            )
        for b in range(NB):
            gcps[b].wait()
            pltpu.async_copy(rows.at[b], accum.at[dstb.at[t, b]], ssb[b],
                             add=True)

    QB = NBATCH // 4

    def body(i, carry):
        j0 = 4 * i
        process(j0, 0, i > 0)
        prefetch(j0 + 3, 3)

        process(j0 + 1, 1, None)

        @pl.when(i < QB - 1)
        def _pf0():
            prefetch(j0 + 4, 0)

        process(j0 + 2, 2, None)

        @pl.when(i < QB - 1)
        def _pf1():
            prefetch(j0 + 5, 1)

        process(j0 + 3, 3, None)

        @pl.when(i < QB - 1)
        def _pf2():
            prefetch(j0 + 6, 2)

        return carry

    lax.fori_loop(0, QB, body, 0)
    for b in range(NB):
        scat_wait(b)
    plsc.subcore_barrier()

    pltpu.sync_copy(
        accum.at[pl.ds(row0, STRIPE)], out_hbm.at[pl.ds(row0, STRIPE)]
    )


def _tc_a_body(x_ref, w_ref, deg_ref, h_ref, g_ref, dinv_ref):
    h = jnp.dot(x_ref[...], w_ref[...], preferred_element_type=jnp.float32)
    dinv = lax.rsqrt(deg_ref[...] + 1.0)
    h_ref[...] = h
    g_ref[...] = h * dinv
    dinv_ref[...] = dinv


_tc_a = pl.pallas_call(
    _tc_a_body,
    out_shape=(
        jax.ShapeDtypeStruct((N, D), jnp.float32),
        jax.ShapeDtypeStruct((N, D), jnp.float32),
        jax.ShapeDtypeStruct((N, 1), jnp.float32),
    ),
)


def _tc_mid_body(p_ref, h_ref, dinv_ref, b_ref, w_ref, h2_ref, g2_ref, y_ref):
    dinv = dinv_ref[...]
    pre = dinv * (p_ref[:N, :] + dinv * h_ref[...]) + b_ref[...][None, :]
    y_ref[...] = pre
    z = jnp.maximum(pre, 0.0)
    h2 = jnp.dot(z, w_ref[...], preferred_element_type=jnp.float32)
    h2_ref[...] = h2
    g2_ref[...] = h2 * dinv


_tc_mid = pl.pallas_call(
    _tc_mid_body,
    out_shape=(
        jax.ShapeDtypeStruct((N, D), jnp.float32),
        jax.ShapeDtypeStruct((N, D), jnp.float32),
        jax.ShapeDtypeStruct((N, D), jnp.float32),
    ),
)


@jax.jit
def kernel(x, edge_index, W1, b1, W2, b2):
    ei = edge_index.astype(jnp.int32)
    src4 = ei[0].reshape(NS, NBATCH, NB, K)
    dst4 = ei[1].reshape(NS, NBATCH, NB, K)
    zeros1 = jnp.zeros((N,), jnp.float32)
    zeros2 = jnp.zeros((NPAD, D), jnp.float32)

    ones = jnp.ones((K,), jnp.float32)
    deg = _deg_kernel(dst4, zeros1, ones)
    degc = deg[:, None]

    h1, g1, dinv = _tc_a(x, W1, degc)

    def body(carry, wb):
        h, g = carry
        w, b = wb
        p = _agg_kernel(g, src4, dst4, zeros2)
        h2, g2, y = _tc_mid(p, h, dinv, b, w)
        return (h2, g2), y

    _, ys = lax.scan(body, (h1, g1), (jnp.stack([W2, W2]), jnp.stack([b1, b2])))
    return ys[1]


# speedup vs baseline: 18.7349x; 1.0277x over previous
import functools

import jax
import jax.numpy as jnp
from jax import lax
from jax.experimental import pallas as pl
from jax.experimental.pallas import tpu as pltpu
from jax.experimental.pallas import tpu_sc as plsc

N = 10000
E = 320000
D = 128

NS = 16
EW = E // NS
K = 40
NB = 5
NBATCH = EW // (NB * K)
HALF = NBATCH // 2
STRIPE = 632
NPAD = NS * STRIPE

_mesh = plsc.VectorSubcoreMesh(
    core_axis_name="c", subcore_axis_name="s", num_cores=1, num_subcores=NS
)


@functools.partial(
    pl.kernel,
    out_type=jax.ShapeDtypeStruct((N,), jnp.float32),
    mesh=_mesh,
    scratch_types=[
        pltpu.VMEM((4, NB, K), jnp.int32),
        pltpu.VMEM((K,), jnp.float32),
        pltpu.VMEM_SHARED((N,), jnp.float32),
        pltpu.SemaphoreType.DMA,
        pltpu.SemaphoreType.DMA,
        pltpu.SemaphoreType.DMA,
        pltpu.SemaphoreType.DMA,
        pltpu.SemaphoreType.DMA,
        pltpu.SemaphoreType.DMA,
        pltpu.SemaphoreType.DMA,
        pltpu.SemaphoreType.DMA,
        pltpu.SemaphoreType.DMA,
    ],
)
def _deg_kernel(dst_hbm, zeros1_hbm, ones_hbm, out_hbm, dstb, onesv, accum,
                si0, si1, si2, si3, ss0, ss1, ss2, ss3, ss4):
    s = lax.axis_index("s")
    si = (si0, si1, si2, si3)
    ssb = (ss0, ss1, ss2, ss3, ss4)

    @pl.when(s == 0)
    def _zero():
        pltpu.sync_copy(zeros1_hbm, accum)

    pltpu.sync_copy(ones_hbm, onesv)
    for t in range(3):
        pltpu.async_copy(dst_hbm.at[s, t], dstb.at[t], si[t])
    plsc.subcore_barrier()

    def scat_wait(b):
        pltpu.make_async_copy(
            onesv, accum.at[dstb.at[0, b]], ssb[b]
        ).wait()

    def process(j, t, wait_pred):
        pltpu.make_async_copy(dst_hbm.at[s, j], dstb.at[t], si[t]).wait()
        for b in range(NB):
            if wait_pred is None:
                scat_wait(b)
            else:
                @pl.when(wait_pred)
                def _w(b=b):
                    scat_wait(b)
            pltpu.async_copy(onesv, accum.at[dstb.at[t, b]], ssb[b],
                             add=True)

    QB = NBATCH // 4

    def body(i, carry):
        j0 = 4 * i
        process(j0, 0, i > 0)
        pltpu.async_copy(dst_hbm.at[s, j0 + 3], dstb.at[3], si[3])

        process(j0 + 1, 1, None)

        @pl.when(i < QB - 1)
        def _pf0():
            pltpu.async_copy(dst_hbm.at[s, j0 + 4], dstb.at[0], si[0])

        process(j0 + 2, 2, None)

        @pl.when(i < QB - 1)
        def _pf1():
            pltpu.async_copy(dst_hbm.at[s, j0 + 5], dstb.at[1], si[1])

        process(j0 + 3, 3, None)

        @pl.when(i < QB - 1)
        def _pf2():
            pltpu.async_copy(dst_hbm.at[s, j0 + 6], dstb.at[2], si[2])

        return carry

    lax.fori_loop(0, QB, body, 0)
    for b in range(NB):
        scat_wait(b)
    plsc.subcore_barrier()

    @pl.when(s == 0)
    def _flush():
        pltpu.sync_copy(accum, out_hbm)


@functools.partial(
    pl.kernel,
    out_type=jax.ShapeDtypeStruct((NPAD, D), jnp.float32),
    mesh=_mesh,
    scratch_types=[
        pltpu.VMEM((4, NB, K), jnp.int32),
        pltpu.VMEM((4, NB, K), jnp.int32),
        pltpu.VMEM((NB, K, D), jnp.float32),
        pltpu.VMEM_SHARED((NPAD, D), jnp.float32),
        pltpu.SemaphoreType.DMA,
        pltpu.SemaphoreType.DMA,
        pltpu.SemaphoreType.DMA,
        pltpu.SemaphoreType.DMA,
        pltpu.SemaphoreType.DMA,
        pltpu.SemaphoreType.DMA,
        pltpu.SemaphoreType.DMA,
        pltpu.SemaphoreType.DMA,
        pltpu.SemaphoreType.DMA,
        pltpu.SemaphoreType.DMA,
        pltpu.SemaphoreType.DMA,
        pltpu.SemaphoreType.DMA,
        pltpu.SemaphoreType.DMA,
        pltpu.SemaphoreType.DMA,
        pltpu.SemaphoreType.DMA,
    ],
)
def _agg_kernel(g_hbm, src_hbm, dst_hbm, zeros2_hbm, out_hbm,
                srcb, dstb, rows, accum,
                semz, si0, si1, si2, si3, sg0, sg1, sg2, sg3, sg4,
                ss0, ss1, ss2, ss3, ss4):
    s = lax.axis_index("s")
    si = (si0, si1, si2, si3)
    sg = (sg0, sg1, sg2, sg3, sg4)
    ssb = (ss0, ss1, ss2, ss3, ss4)
    row0 = s * STRIPE

    zcp = pltpu.async_copy(
        zeros2_hbm.at[pl.ds(row0, STRIPE)], accum.at[pl.ds(row0, STRIPE)], semz
    )
    for t in range(3):
        pltpu.async_copy(src_hbm.at[s, t], srcb.at[t], si[t])
        pltpu.async_copy(dst_hbm.at[s, t], dstb.at[t], si[t])
    zcp.wait()
    plsc.subcore_barrier()

    def scat_wait(b):
        pltpu.make_async_copy(
            rows.at[b], accum.at[dstb.at[0, b]], ssb[b]
        ).wait()

    def prefetch(j, t):
        pltpu.async_copy(src_hbm.at[s, j], srcb.at[t], si[t])
        pltpu.async_copy(dst_hbm.at[s, j], dstb.at[t], si[t])

    def process(j, t, wait_pred):
        pltpu.make_async_copy(src_hbm.at[s, j], srcb.at[t], si[t]).wait()
        pltpu.make_async_copy(dst_hbm.at[s, j], dstb.at[t], si[t]).wait()
        gcps = []
        for b in range(NB):
            if wait_pred is None:
                scat_wait(b)
            else:
                @pl.when(wait_pred)
                def _w(b=b):
                    scat_wait(b)
            gcps.append(
                pltpu.async_copy(g_hbm.at[srcb.at[t, b]], rows.at[b], sg[b])
            )
        for b in range(NB):
            gcps[b].wait()
            pltpu.async_copy(rows.at[b], accum.at[dstb.at[t, b]], ssb[b],
                             add=True)

    QB = NBATCH // 4

    def body(i, carry):
        j0 = 4 * i
        process(j0, 0, i > 0)
        prefetch(j0 + 3, 3)

        process(j0 + 1, 1, None)

        @pl.when(i < QB - 1)
        def _pf0():
            prefetch(j0 + 4, 0)

        process(j0 + 2, 2, None)

        @pl.when(i < QB - 1)
        def _pf1():
            prefetch(j0 + 5, 1)

        process(j0 + 3, 3, None)

        @pl.when(i < QB - 1)
        def _pf2():
            prefetch(j0 + 6, 2)

        return carry

    lax.fori_loop(0, QB, body, 0)
    for b in range(NB):
        scat_wait(b)
    plsc.subcore_barrier()

    pltpu.sync_copy(
        accum.at[pl.ds(row0, STRIPE)], out_hbm.at[pl.ds(row0, STRIPE)]
    )


def _tc_a_body(x_ref, w_ref, deg_ref, h_ref, g_ref, dinv_ref):
    h = jnp.dot(x_ref[...], w_ref[...], preferred_element_type=jnp.float32)
    dinv = lax.rsqrt(deg_ref[...] + 1.0)
    h_ref[...] = h
    g_ref[...] = h * dinv
    dinv_ref[...] = dinv


_tc_a = pl.pallas_call(
    _tc_a_body,
    out_shape=(
        jax.ShapeDtypeStruct((N, D), jnp.float32),
        jax.ShapeDtypeStruct((N, D), jnp.float32),
        jax.ShapeDtypeStruct((N, 1), jnp.float32),
    ),
)


def _tc_mid_body(p_ref, h_ref, dinv_ref, b_ref, w_ref, h2_ref, g2_ref, y_ref):
    dinv = dinv_ref[...]
    pre = dinv * (p_ref[:N, :] + dinv * h_ref[...]) + b_ref[...][None, :]
    y_ref[...] = pre
    z = jnp.maximum(pre, 0.0)
    h2 = jnp.dot(z, w_ref[...], preferred_element_type=jnp.float32)
    h2_ref[...] = h2
    g2_ref[...] = h2 * dinv


_tc_mid = pl.pallas_call(
    _tc_mid_body,
    out_shape=(
        jax.ShapeDtypeStruct((N, D), jnp.float32),
        jax.ShapeDtypeStruct((N, D), jnp.float32),
        jax.ShapeDtypeStruct((N, D), jnp.float32),
    ),
)


@jax.jit
def kernel(x, edge_index, W1, b1, W2, b2):
    ei = edge_index.astype(jnp.int32)
    src4 = ei[0].reshape(NS, NBATCH, NB, K)
    dst4 = ei[1].reshape(NS, NBATCH, NB, K)
    zeros1 = jnp.zeros((N,), jnp.float32)
    zeros2 = jnp.zeros((NPAD, D), jnp.float32)

    ones = jnp.ones((K,), jnp.float32)
    deg = _deg_kernel(dst4, zeros1, ones)
    degc = deg[:, None]

    h1, g1, dinv = _tc_a(x, W1, degc)

    def body(carry, wb):
        h, g = carry
        w, b = wb
        p = _agg_kernel(g, src4, dst4, zeros2)
        h2, g2, y = _tc_mid(p, h, dinv, b, w)
        return (h2, g2), y

    _, ys = lax.scan(body, (h1, g1), (jnp.stack([W2, W2]), jnp.stack([b1, b2])))
    return ys[1]


# speedup vs baseline: 28.8166x; 1.5381x over previous
import functools

import jax
import jax.numpy as jnp
from jax import lax
from jax.experimental import pallas as pl
from jax.experimental.pallas import tpu as pltpu
from jax.experimental.pallas import tpu_sc as plsc

N = 10000
E = 320000
D = 128

NC = 2
NS = 16
NW = NC * NS
EW = E // NW
K = 40
NB = 5
NBATCH = EW // (NB * K)
QB = NBATCH // 5
STRIPE = 632
NPAD = NS * STRIPE

_mesh = plsc.VectorSubcoreMesh(
    core_axis_name="c", subcore_axis_name="s", num_cores=NC, num_subcores=NS
)


@functools.partial(
    pl.kernel,
    out_type=(
        jax.ShapeDtypeStruct((N,), jnp.float32),
        jax.ShapeDtypeStruct((N,), jnp.float32),
    ),
    mesh=_mesh,
    scratch_types=[
        pltpu.VMEM((5, NB, K), jnp.int32),
        pltpu.VMEM((K,), jnp.float32),
        pltpu.VMEM_SHARED((N,), jnp.float32),
        pltpu.SemaphoreType.DMA,
        pltpu.SemaphoreType.DMA,
        pltpu.SemaphoreType.DMA,
        pltpu.SemaphoreType.DMA,
        pltpu.SemaphoreType.DMA,
        pltpu.SemaphoreType.DMA,
        pltpu.SemaphoreType.DMA,
        pltpu.SemaphoreType.DMA,
        pltpu.SemaphoreType.DMA,
        pltpu.SemaphoreType.DMA,
    ],
)
def _deg_kernel(dst_hbm, zeros1_hbm, ones_hbm, out0_hbm, out1_hbm,
                dstb, onesv, accum,
                si0, si1, si2, si3, si4, ss0, ss1, ss2, ss3, ss4):
    c = lax.axis_index("c")
    s = lax.axis_index("s")
    w = c * NS + s
    si = (si0, si1, si2, si3, si4)
    ssb = (ss0, ss1, ss2, ss3, ss4)

    @pl.when(s == 0)
    def _zero():
        pltpu.sync_copy(zeros1_hbm, accum)

    pltpu.sync_copy(ones_hbm, onesv)
    for t in range(4):
        pltpu.async_copy(dst_hbm.at[w, t], dstb.at[t], si[t])
    plsc.subcore_barrier()

    def scat_wait(b):
        pltpu.make_async_copy(onesv, accum.at[dstb.at[0, b]], ssb[b]).wait()

    def process(j, t, wait_pred):
        pltpu.make_async_copy(dst_hbm.at[w, j], dstb.at[t], si[t]).wait()
        for b in range(NB):
            if wait_pred is None:
                scat_wait(b)
            else:
                @pl.when(wait_pred)
                def _w(b=b):
                    scat_wait(b)
            pltpu.async_copy(onesv, accum.at[dstb.at[t, b]], ssb[b],
                             add=True)

    def prefetch(j, t):
        pltpu.async_copy(dst_hbm.at[w, j], dstb.at[t], si[t])

    def body(i, carry):
        j0 = 5 * i
        process(j0, 0, i > 0)
        prefetch(j0 + 4, 4)
        for k in range(1, 5):
            process(j0 + k, k, None)
            if k < 4:
                @pl.when(i < QB - 1)
                def _pf(k=k):
                    prefetch(j0 + 4 + k, k - 1)
        @pl.when(i < QB - 1)
        def _pf3():
            prefetch(j0 + 8, 3)
        return carry

    lax.fori_loop(0, QB, body, 0)
    for b in range(NB):
        scat_wait(b)
    plsc.subcore_barrier()

    @pl.when(jnp.logical_and(s == 0, c == 0))
    def _flush0():
        pltpu.sync_copy(accum, out0_hbm)

    @pl.when(jnp.logical_and(s == 0, c == 1))
    def _flush1():
        pltpu.sync_copy(accum, out1_hbm)


@functools.partial(
    pl.kernel,
    out_type=jax.ShapeDtypeStruct((NC, NPAD, D), jnp.float32),
    mesh=_mesh,
    scratch_types=[
        pltpu.VMEM((5, NB, K), jnp.int32),
        pltpu.VMEM((5, NB, K), jnp.int32),
        pltpu.VMEM((NB, K, D), jnp.float32),
        pltpu.VMEM_SHARED((NPAD, D), jnp.float32),
        pltpu.SemaphoreType.DMA,
        pltpu.SemaphoreType.DMA,
        pltpu.SemaphoreType.DMA,
        pltpu.SemaphoreType.DMA,
        pltpu.SemaphoreType.DMA,
        pltpu.SemaphoreType.DMA,
        pltpu.SemaphoreType.DMA,
        pltpu.SemaphoreType.DMA,
        pltpu.SemaphoreType.DMA,
        pltpu.SemaphoreType.DMA,
        pltpu.SemaphoreType.DMA,
        pltpu.SemaphoreType.DMA,
        pltpu.SemaphoreType.DMA,
        pltpu.SemaphoreType.DMA,
        pltpu.SemaphoreType.DMA,
        pltpu.SemaphoreType.DMA,
    ],
)
def _agg_kernel(g_hbm, src_hbm, dst_hbm, zeros2_hbm, out_hbm,
                srcb, dstb, rows, accum,
                semz, si0, si1, si2, si3, si4, sg0, sg1, sg2, sg3, sg4,
                ss0, ss1, ss2, ss3, ss4):
    c = lax.axis_index("c")
    s = lax.axis_index("s")
    w = c * NS + s
    si = (si0, si1, si2, si3, si4)
    sg = (sg0, sg1, sg2, sg3, sg4)
    ssb = (ss0, ss1, ss2, ss3, ss4)
    row0 = s * STRIPE

    zcp = pltpu.async_copy(
        zeros2_hbm.at[pl.ds(row0, STRIPE)], accum.at[pl.ds(row0, STRIPE)], semz
    )
    for t in range(4):
        pltpu.async_copy(src_hbm.at[w, t], srcb.at[t], si[t])
        pltpu.async_copy(dst_hbm.at[w, t], dstb.at[t], si[t])
    zcp.wait()
    plsc.subcore_barrier()

    def scat_wait(b):
        pltpu.make_async_copy(
            rows.at[b], accum.at[dstb.at[0, b]], ssb[b]
        ).wait()

    def prefetch(j, t):
        pltpu.async_copy(src_hbm.at[w, j], srcb.at[t], si[t])
        pltpu.async_copy(dst_hbm.at[w, j], dstb.at[t], si[t])

    def process(j, t, wait_pred):
        pltpu.make_async_copy(src_hbm.at[w, j], srcb.at[t], si[t]).wait()
        pltpu.make_async_copy(dst_hbm.at[w, j], dstb.at[t], si[t]).wait()
        gcps = []
        for b in range(NB):
            if wait_pred is None:
                scat_wait(b)
            else:
                @pl.when(wait_pred)
                def _w(b=b):
                    scat_wait(b)
            gcps.append(
                pltpu.async_copy(g_hbm.at[srcb.at[t, b]], rows.at[b], sg[b])
            )
        for b in range(NB):
            gcps[b].wait()
            pltpu.async_copy(rows.at[b], accum.at[dstb.at[t, b]], ssb[b],
                             add=True)

    def body(i, carry):
        j0 = 5 * i
        process(j0, 0, i > 0)
        prefetch(j0 + 4, 4)
        for k in range(1, 5):
            process(j0 + k, k, None)
            if k < 4:
                @pl.when(i < QB - 1)
                def _pf(k=k):
                    prefetch(j0 + 4 + k, k - 1)
        @pl.when(i < QB - 1)
        def _pf3():
            prefetch(j0 + 8, 3)
        return carry

    lax.fori_loop(0, QB, body, 0)
    for b in range(NB):
        scat_wait(b)
    plsc.subcore_barrier()

    pltpu.sync_copy(
        accum.at[pl.ds(row0, STRIPE)], out_hbm.at[c, pl.ds(row0, STRIPE)]
    )


def _tc_a_body(x_ref, w_ref, d0_ref, d1_ref, h_ref, g_ref, dinv_ref):
    h = jnp.dot(x_ref[...], w_ref[...], preferred_element_type=jnp.float32)
    dinv = lax.rsqrt(d0_ref[...] + d1_ref[...] + 1.0)
    h_ref[...] = h
    g_ref[...] = h * dinv
    dinv_ref[...] = dinv


_tc_a = pl.pallas_call(
    _tc_a_body,
    out_shape=(
        jax.ShapeDtypeStruct((N, D), jnp.float32),
        jax.ShapeDtypeStruct((N, D), jnp.float32),
        jax.ShapeDtypeStruct((N, 1), jnp.float32),
    ),
)


def _tc_mid_body(p_ref, h_ref, dinv_ref, b_ref, w_ref, h2_ref, g2_ref, y_ref):
    dinv = dinv_ref[...]
    agg = p_ref[0, :N, :] + p_ref[1, :N, :]
    pre = dinv * (agg + dinv * h_ref[...]) + b_ref[...][None, :]
    y_ref[...] = pre
    z = jnp.maximum(pre, 0.0)
    h2 = jnp.dot(z, w_ref[...], preferred_element_type=jnp.float32)
    h2_ref[...] = h2
    g2_ref[...] = h2 * dinv


_tc_mid = pl.pallas_call(
    _tc_mid_body,
    out_shape=(
        jax.ShapeDtypeStruct((N, D), jnp.float32),
        jax.ShapeDtypeStruct((N, D), jnp.float32),
        jax.ShapeDtypeStruct((N, D), jnp.float32),
    ),
)


@jax.jit
def kernel(x, edge_index, W1, b1, W2, b2):
    ei = edge_index.astype(jnp.int32)
    src4 = ei[0].reshape(NW, NBATCH, NB, K)
    dst4 = ei[1].reshape(NW, NBATCH, NB, K)
    zeros1 = jnp.zeros((N,), jnp.float32)
    zeros2 = jnp.zeros((NPAD, D), jnp.float32)
    ones = jnp.ones((K,), jnp.float32)

    d0, d1 = _deg_kernel(dst4, zeros1, ones)

    h1, g1, dinv = _tc_a(x, W1, d0[:, None], d1[:, None])

    def body(carry, wb):
        h, g = carry
        w, b = wb
        p = _agg_kernel(g, src4, dst4, zeros2)
        h2, g2, y = _tc_mid(p, h, dinv, b, w)
        return (h2, g2), y

    _, ys = lax.scan(body, (h1, g1), (jnp.stack([W2, W2]), jnp.stack([b1, b2])))
    return ys[1]


# speedup vs baseline: 29.1569x; 1.0118x over previous
import functools

import jax
import jax.numpy as jnp
from jax import lax
from jax.experimental import pallas as pl
from jax.experimental.pallas import tpu as pltpu
from jax.experimental.pallas import tpu_sc as plsc

N = 10000
E = 320000
D = 128

NC = 2
NS = 16
NW = NC * NS
EW = E // NW
K = 50
NB = 5
NBATCH = EW // (NB * K)
QB = NBATCH // 5
STRIPE = 632
NPAD = NS * STRIPE

_mesh = plsc.VectorSubcoreMesh(
    core_axis_name="c", subcore_axis_name="s", num_cores=NC, num_subcores=NS
)


@functools.partial(
    pl.kernel,
    out_type=(
        jax.ShapeDtypeStruct((N,), jnp.float32),
        jax.ShapeDtypeStruct((N,), jnp.float32),
    ),
    mesh=_mesh,
    scratch_types=[
        pltpu.VMEM((5, NB, K), jnp.int32),
        pltpu.VMEM((K,), jnp.float32),
        pltpu.VMEM_SHARED((N,), jnp.float32),
        pltpu.SemaphoreType.DMA,
        pltpu.SemaphoreType.DMA,
        pltpu.SemaphoreType.DMA,
        pltpu.SemaphoreType.DMA,
        pltpu.SemaphoreType.DMA,
        pltpu.SemaphoreType.DMA,
        pltpu.SemaphoreType.DMA,
        pltpu.SemaphoreType.DMA,
        pltpu.SemaphoreType.DMA,
        pltpu.SemaphoreType.DMA,
    ],
)
def _deg_kernel(dst_hbm, zeros1_hbm, ones_hbm, out0_hbm, out1_hbm,
                dstb, onesv, accum,
                si0, si1, si2, si3, si4, ss0, ss1, ss2, ss3, ss4):
    c = lax.axis_index("c")
    s = lax.axis_index("s")
    w = c * NS + s
    si = (si0, si1, si2, si3, si4)
    ssb = (ss0, ss1, ss2, ss3, ss4)

    @pl.when(s == 0)
    def _zero():
        pltpu.sync_copy(zeros1_hbm, accum)

    pltpu.sync_copy(ones_hbm, onesv)
    for t in range(4):
        pltpu.async_copy(dst_hbm.at[w, t], dstb.at[t], si[t])
    plsc.subcore_barrier()

    def scat_wait(b):
        pltpu.make_async_copy(onesv, accum.at[dstb.at[0, b]], ssb[b]).wait()

    def process(j, t, wait_pred):
        pltpu.make_async_copy(dst_hbm.at[w, j], dstb.at[t], si[t]).wait()
        for b in range(NB):
            if wait_pred is None:
                scat_wait(b)
            else:
                @pl.when(wait_pred)
                def _w(b=b):
                    scat_wait(b)
            pltpu.async_copy(onesv, accum.at[dstb.at[t, b]], ssb[b],
                             add=True)

    def prefetch(j, t):
        pltpu.async_copy(dst_hbm.at[w, j], dstb.at[t], si[t])

    def body(i, carry):
        j0 = 5 * i
        process(j0, 0, i > 0)
        prefetch(j0 + 4, 4)
        for k in range(1, 5):
            process(j0 + k, k, None)
            if k < 4:
                @pl.when(i < QB - 1)
                def _pf(k=k):
                    prefetch(j0 + 4 + k, k - 1)
        @pl.when(i < QB - 1)
        def _pf3():
            prefetch(j0 + 8, 3)
        return carry

    lax.fori_loop(0, QB, body, 0)
    for b in range(NB):
        scat_wait(b)
    plsc.subcore_barrier()

    @pl.when(jnp.logical_and(s == 0, c == 0))
    def _flush0():
        pltpu.sync_copy(accum, out0_hbm)

    @pl.when(jnp.logical_and(s == 0, c == 1))
    def _flush1():
        pltpu.sync_copy(accum, out1_hbm)


@functools.partial(
    pl.kernel,
    out_type=jax.ShapeDtypeStruct((NC, NPAD, D), jnp.float32),
    mesh=_mesh,
    scratch_types=[
        pltpu.VMEM((5, NB, K), jnp.int32),
        pltpu.VMEM((5, NB, K), jnp.int32),
        pltpu.VMEM((NB, K, D), jnp.float32),
        pltpu.VMEM_SHARED((NPAD, D), jnp.float32),
        pltpu.SemaphoreType.DMA,
        pltpu.SemaphoreType.DMA,
        pltpu.SemaphoreType.DMA,
        pltpu.SemaphoreType.DMA,
        pltpu.SemaphoreType.DMA,
        pltpu.SemaphoreType.DMA,
        pltpu.SemaphoreType.DMA,
        pltpu.SemaphoreType.DMA,
        pltpu.SemaphoreType.DMA,
        pltpu.SemaphoreType.DMA,
        pltpu.SemaphoreType.DMA,
        pltpu.SemaphoreType.DMA,
        pltpu.SemaphoreType.DMA,
        pltpu.SemaphoreType.DMA,
        pltpu.SemaphoreType.DMA,
        pltpu.SemaphoreType.DMA,
    ],
)
def _agg_kernel(g_hbm, src_hbm, dst_hbm, zeros2_hbm, out_hbm,
                srcb, dstb, rows, accum,
                semz, si0, si1, si2, si3, si4, sg0, sg1, sg2, sg3, sg4,
                ss0, ss1, ss2, ss3, ss4):
    c = lax.axis_index("c")
    s = lax.axis_index("s")
    w = c * NS + s
    si = (si0, si1, si2, si3, si4)
    sg = (sg0, sg1, sg2, sg3, sg4)
    ssb = (ss0, ss1, ss2, ss3, ss4)
    row0 = s * STRIPE

    zcp = pltpu.async_copy(
        zeros2_hbm.at[pl.ds(row0, STRIPE)], accum.at[pl.ds(row0, STRIPE)], semz
    )
    for t in range(4):
        pltpu.async_copy(src_hbm.at[w, t], srcb.at[t], si[t])
        pltpu.async_copy(dst_hbm.at[w, t], dstb.at[t], si[t])
    zcp.wait()
    plsc.subcore_barrier()

    def scat_wait(b):
        pltpu.make_async_copy(
            rows.at[b], accum.at[dstb.at[0, b]], ssb[b]
        ).wait()

    def prefetch(j, t):
        pltpu.async_copy(src_hbm.at[w, j], srcb.at[t], si[t])
        pltpu.async_copy(dst_hbm.at[w, j], dstb.at[t], si[t])

    def process(j, t, wait_pred):
        pltpu.make_async_copy(src_hbm.at[w, j], srcb.at[t], si[t]).wait()
        pltpu.make_async_copy(dst_hbm.at[w, j], dstb.at[t], si[t]).wait()
        gcps = []
        for b in range(NB):
            if wait_pred is None:
                scat_wait(b)
            else:
                @pl.when(wait_pred)
                def _w(b=b):
                    scat_wait(b)
            gcps.append(
                pltpu.async_copy(g_hbm.at[srcb.at[t, b]], rows.at[b], sg[b])
            )
        for b in range(NB):
            gcps[b].wait()
            pltpu.async_copy(rows.at[b], accum.at[dstb.at[t, b]], ssb[b],
                             add=True)

    def body(i, carry):
        j0 = 5 * i
        process(j0, 0, i > 0)
        prefetch(j0 + 4, 4)
        for k in range(1, 5):
            process(j0 + k, k, None)
            if k < 4:
                @pl.when(i < QB - 1)
                def _pf(k=k):
                    prefetch(j0 + 4 + k, k - 1)
        @pl.when(i < QB - 1)
        def _pf3():
            prefetch(j0 + 8, 3)
        return carry

    lax.fori_loop(0, QB, body, 0)
    for b in range(NB):
        scat_wait(b)
    plsc.subcore_barrier()

    pltpu.sync_copy(
        accum.at[pl.ds(row0, STRIPE)], out_hbm.at[c, pl.ds(row0, STRIPE)]
    )


def _tc_a_body(x_ref, w_ref, d0_ref, d1_ref, h_ref, g_ref, dinv_ref):
    h = jnp.dot(x_ref[...], w_ref[...], preferred_element_type=jnp.float32)
    dinv = lax.rsqrt(d0_ref[...] + d1_ref[...] + 1.0)
    h_ref[...] = h
    g_ref[...] = h * dinv
    dinv_ref[...] = dinv


_tc_a = pl.pallas_call(
    _tc_a_body,
    out_shape=(
        jax.ShapeDtypeStruct((N, D), jnp.float32),
        jax.ShapeDtypeStruct((N, D), jnp.float32),
        jax.ShapeDtypeStruct((N, 1), jnp.float32),
    ),
)


def _tc_mid_body(p_ref, h_ref, dinv_ref, b_ref, w_ref, h2_ref, g2_ref, y_ref):
    dinv = dinv_ref[...]
    agg = p_ref[0, :N, :] + p_ref[1, :N, :]
    pre = dinv * (agg + dinv * h_ref[...]) + b_ref[...][None, :]
    y_ref[...] = pre
    z = jnp.maximum(pre, 0.0)
    h2 = jnp.dot(z, w_ref[...], preferred_element_type=jnp.float32)
    h2_ref[...] = h2
    g2_ref[...] = h2 * dinv


_tc_mid = pl.pallas_call(
    _tc_mid_body,
    out_shape=(
        jax.ShapeDtypeStruct((N, D), jnp.float32),
        jax.ShapeDtypeStruct((N, D), jnp.float32),
        jax.ShapeDtypeStruct((N, D), jnp.float32),
    ),
)


@jax.jit
def kernel(x, edge_index, W1, b1, W2, b2):
    ei = edge_index.astype(jnp.int32)
    src4 = ei[0].reshape(NW, NBATCH, NB, K)
    dst4 = ei[1].reshape(NW, NBATCH, NB, K)
    zeros1 = jnp.zeros((N,), jnp.float32)
    zeros2 = jnp.zeros((NPAD, D), jnp.float32)
    ones = jnp.ones((K,), jnp.float32)

    d0, d1 = _deg_kernel(dst4, zeros1, ones)

    h1, g1, dinv = _tc_a(x, W1, d0[:, None], d1[:, None])

    def body(carry, wb):
        h, g = carry
        w, b = wb
        p = _agg_kernel(g, src4, dst4, zeros2)
        h2, g2, y = _tc_mid(p, h, dinv, b, w)
        return (h2, g2), y

    _, ys = lax.scan(body, (h1, g1), (jnp.stack([W2, W2]), jnp.stack([b1, b2])))
    return ys[1]
